# baseline (speedup 1.0000x reference)
import jax
import jax.numpy as jnp
from jax import lax
from jax.experimental import pallas as pl
from jax.experimental.pallas import tpu as pltpu
from jax.experimental.pallas import tpu_sc as plsc

B, N, D = 8, 2048, 128
NUM_CORES, NUM_SUBCORES = 2, 16
NW = NUM_CORES * NUM_SUBCORES
SC_ROWS = 4
SUBC_PER_ROW = NW // SC_ROWS
SB_P = 64
SB_F = SB_P * D
NSB = (N // SB_P) // SUBC_PER_ROW
ROW_F = N * D
VEC = 16


def _sc_body(x_hbm, off_hbm, out_hbm, buf, zbuf, offb, sem_off, sem_in, sem_out):
    c = lax.axis_index("c")
    s = lax.axis_index("s")
    wid = c * NUM_SUBCORES + s
    b = wid % SC_ROWS
    q = wid // SC_ROWS
    row_base = b * ROW_F

    def sb_pos(k):
        return (q + SUBC_PER_ROW * k) * SB_P

    for k in range(NSB):
        pltpu.async_copy(
            x_hbm.at[pl.ds(row_base + sb_pos(k) * D, SB_F)],
            buf.at[pl.ds(k * SB_F, SB_F)],
            sem_in.at[k],
        )

    off_copy = pltpu.make_async_copy(off_hbm, offb.at[pl.ds(0, B + 1)], sem_off)
    off_copy.start()

    zero = jnp.zeros((VEC,), jnp.float32)

    def zfill(p, carry):
        for u in range(D // VEC):
            zbuf[pl.ds(p * D + u * VEC, VEC)] = zero
        return carry

    lax.fori_loop(0, SB_P, zfill, 0)

    off_copy.wait()
    offv = offb[pl.ds(b, VEC)]
    nv = jnp.clip(offv[1] - offv[0], 0, N)

    for k in range(NSB):
        @pl.when(sb_pos(k) >= nv)
        def _(k=k):
            pltpu.async_copy(
                zbuf, out_hbm.at[pl.ds(row_base + sb_pos(k) * D, SB_F)], sem_out
            )

    for k in range(NSB):
        @pl.when(sb_pos(k) < nv)
        def _(k=k):
            pltpu.make_async_copy(
                x_hbm.at[pl.ds(row_base + sb_pos(k) * D, SB_F)],
                buf.at[pl.ds(k * SB_F, SB_F)],
                sem_in.at[k],
            ).wait()

            nvk = jnp.minimum(nv - sb_pos(k), SB_P)

            def ztail(p, carry):
                for u in range(D // VEC):
                    buf[pl.ds(k * SB_F + p * D + u * VEC, VEC)] = zero
                return carry

            lax.fori_loop(nvk, SB_P, ztail, 0)

            pltpu.async_copy(
                buf.at[pl.ds(k * SB_F, SB_F)],
                out_hbm.at[pl.ds(row_base + sb_pos(k) * D, SB_F)],
                sem_out,
            )

    for k in range(NSB):
        pltpu.make_async_copy(
            zbuf, out_hbm.at[pl.ds(0, SB_F)], sem_out
        ).wait()

        @pl.when(sb_pos(k) >= nv)
        def _(k=k):
            pltpu.make_async_copy(
                x_hbm.at[pl.ds(row_base + sb_pos(k) * D, SB_F)],
                buf.at[pl.ds(k * SB_F, SB_F)],
                sem_in.at[k],
            ).wait()


def _tc_body(off_sref, x_ref, y_in_ref, y_ref):
    del y_in_ref
    i = pl.program_id(0)
    b = i + SC_ROWS
    ln = off_sref[b + 1] - off_sref[b]
    ids = lax.broadcasted_iota(jnp.int32, (1, N, D), 1)
    y_ref[...] = jnp.where(ids < ln, x_ref[...], 0.0)


def kernel(x, x_offsets, all_timestamps, invalid_attn_mask):
    del all_timestamps, invalid_attn_mask
    xf = x.reshape(-1)
    off = x_offsets.astype(jnp.int32)
    mesh = plsc.VectorSubcoreMesh(core_axis_name="c", subcore_axis_name="s")
    sc_fn = pl.kernel(
        _sc_body,
        mesh=mesh,
        out_type=jax.ShapeDtypeStruct((B * N * D,), jnp.float32),
        scratch_types=[
            pltpu.VMEM((NSB * SB_F,), jnp.float32),
            pltpu.VMEM((SB_F,), jnp.float32),
            pltpu.VMEM((32,), jnp.int32),
            pltpu.SemaphoreType.DMA,
            pltpu.SemaphoreType.DMA((NSB,)),
            pltpu.SemaphoreType.DMA,
        ],
    )
    y_sc = sc_fn(xf, off).reshape(B, N, D)

    row_spec = pl.BlockSpec((1, N, D), lambda i: (i + SC_ROWS, 0, 0))
    y = pl.pallas_call(
        _tc_body,
        grid=(B - SC_ROWS,),
        in_specs=[
            pl.BlockSpec(memory_space=pltpu.SMEM),
            row_spec,
            pl.BlockSpec(memory_space=pl.ANY),
        ],
        out_specs=row_spec,
        out_shape=jax.ShapeDtypeStruct((B, N, D), jnp.float32),
        input_output_aliases={2: 0},
    )(off, x, y_sc)
    return y
